# Initial kernel scaffold; baseline (speedup 1.0000x reference)
import jax
import jax.numpy as jnp
from jax.experimental import pallas as pl


def kernel(h_states, seq_start_end, end_pos, rel_pos, W, b, gamma, beta):
    raise NotImplementedError("write your pallas kernel here")


# speedup vs baseline: 9.0823x; 9.0823x over previous
import functools

import jax
import jax.numpy as jnp
from jax.experimental import pallas as pl

H_DIM = 64
GRID = 8
G2 = GRID * GRID
NBHD = 2.0
BOTTLENECK = 1024
NUM_SEQS = 32
PEDS = 128
BATCH = NUM_SEQS * PEDS
MBLK = 512


def _pool_body(h_ref, xq_ref, yq_ref, xp_ref, yp_ref, out_ref):
    xp = xp_ref[...]
    yp = yp_ref[...]
    xq = xq_ref[0]
    yq = yq_ref[0]
    tlx = xp - NBHD / 2
    tly = yp + NBHD / 2
    brx = xp + NBHD / 2
    bry = yp - NBHD / 2
    cx = jnp.floor((xq - tlx) / NBHD * GRID)
    cy = jnp.floor((tly - yq) / NBHD * GRID)
    cell = (cx + cy * GRID).astype(jnp.int32)
    oob_x = (xq >= brx) | (xq <= tlx)
    oob_y = (yq >= tly) | (yq <= bry)
    pp = jax.lax.broadcasted_iota(jnp.int32, (PEDS, PEDS), 0)
    qq = jax.lax.broadcasted_iota(jnp.int32, (PEDS, PEDS), 1)
    valid = jnp.logical_not(oob_x | oob_y) & (pp != qq)
    c_iota = jax.lax.broadcasted_iota(jnp.int32, (PEDS, G2, PEDS), 1)
    hit = (cell[:, None, :] == c_iota) & valid[:, None, :]
    m = jnp.where(hit, 1.0, 0.0).reshape(PEDS * G2, PEDS)
    out_ref[...] = jnp.dot(m, h_ref[...], preferred_element_type=jnp.float32)


def _proj_body(x_ref, w_ref, y_ref, s_ref, s2_ref):
    i = pl.program_id(0)
    y = jax.lax.dot_general(
        x_ref[...], w_ref[...], (((1,), (1,)), ((), ())),
        preferred_element_type=jnp.float32)
    y_ref[...] = y
    ps = jnp.sum(y, axis=0, keepdims=True)
    ps2 = jnp.sum(y * y, axis=0, keepdims=True)

    @pl.when(i == 0)
    def _():
        s_ref[...] = ps
        s2_ref[...] = ps2

    @pl.when(i != 0)
    def _():
        s_ref[...] += ps
        s2_ref[...] += ps2


def _bn_body(y_ref, s_ref, s2_ref, g_ref, bt_ref, out_ref):
    mean = s_ref[...] * (1.0 / BATCH)
    ex2 = s2_ref[...] * (1.0 / BATCH)
    var = ex2 - mean * mean
    inv = jax.lax.rsqrt(var + 1e-5)
    yn = (y_ref[...] - mean) * inv * g_ref[...] + bt_ref[...]
    out_ref[...] = jnp.maximum(yn, 0.0)


@functools.partial(jax.jit, static_argnames=())
def kernel(h_states, seq_start_end, end_pos, rel_pos, W, b, gamma, beta):
    del seq_start_end, rel_pos
    h_flat = h_states.reshape(BATCH, H_DIM)
    xq = end_pos[:, 0].reshape(NUM_SEQS, 1, PEDS)
    yq = end_pos[:, 1].reshape(NUM_SEQS, 1, PEDS)
    xp = end_pos[:, 0].reshape(BATCH, 1)
    yp = end_pos[:, 1].reshape(BATCH, 1)

    pool = pl.pallas_call(
        _pool_body,
        grid=(NUM_SEQS,),
        in_specs=[
            pl.BlockSpec((PEDS, H_DIM), lambda i: (i, 0)),
            pl.BlockSpec((1, 1, PEDS), lambda i: (i, 0, 0)),
            pl.BlockSpec((1, 1, PEDS), lambda i: (i, 0, 0)),
            pl.BlockSpec((PEDS, 1), lambda i: (i, 0)),
            pl.BlockSpec((PEDS, 1), lambda i: (i, 0)),
        ],
        out_specs=pl.BlockSpec((PEDS * G2, H_DIM), lambda i: (i, 0)),
        out_shape=jax.ShapeDtypeStruct((BATCH * G2, H_DIM), jnp.float32),
    )(h_flat, xq, yq, xp, yp)
    pool_h = pool.reshape(BATCH, G2 * H_DIM)

    y_raw, s, s2 = pl.pallas_call(
        _proj_body,
        grid=(BATCH // MBLK,),
        in_specs=[
            pl.BlockSpec((MBLK, G2 * H_DIM), lambda i: (i, 0)),
            pl.BlockSpec((BOTTLENECK, G2 * H_DIM), lambda i: (0, 0)),
        ],
        out_specs=[
            pl.BlockSpec((MBLK, BOTTLENECK), lambda i: (i, 0)),
            pl.BlockSpec((1, BOTTLENECK), lambda i: (0, 0)),
            pl.BlockSpec((1, BOTTLENECK), lambda i: (0, 0)),
        ],
        out_shape=[
            jax.ShapeDtypeStruct((BATCH, BOTTLENECK), jnp.float32),
            jax.ShapeDtypeStruct((1, BOTTLENECK), jnp.float32),
            jax.ShapeDtypeStruct((1, BOTTLENECK), jnp.float32),
        ],
    )(pool_h, W)

    out = pl.pallas_call(
        _bn_body,
        grid=(BATCH // MBLK,),
        in_specs=[
            pl.BlockSpec((MBLK, BOTTLENECK), lambda i: (i, 0)),
            pl.BlockSpec((1, BOTTLENECK), lambda i: (0, 0)),
            pl.BlockSpec((1, BOTTLENECK), lambda i: (0, 0)),
            pl.BlockSpec((1, BOTTLENECK), lambda i: (0, 0)),
            pl.BlockSpec((1, BOTTLENECK), lambda i: (0, 0)),
        ],
        out_specs=pl.BlockSpec((MBLK, BOTTLENECK), lambda i: (i, 0)),
        out_shape=jax.ShapeDtypeStruct((BATCH, BOTTLENECK), jnp.float32),
    )(y_raw, s, s2, gamma.reshape(1, BOTTLENECK), beta.reshape(1, BOTTLENECK))
    return out


# speedup vs baseline: 10.0113x; 1.1023x over previous
import functools

import jax
import jax.numpy as jnp
from jax.experimental import pallas as pl

H_DIM = 64
GRID = 8
G2 = GRID * GRID
NBHD = 2.0
BOTTLENECK = 1024
NUM_SEQS = 32
PEDS = 128
BATCH = NUM_SEQS * PEDS
MBLK = 512


def _pool_body(h_ref, xq_ref, yq_ref, xp_ref, yp_ref, out_ref):
    xp = xp_ref[...]
    yp = yp_ref[...]
    xq = xq_ref[0]
    yq = yq_ref[0]
    tlx = xp - NBHD / 2
    tly = yp + NBHD / 2
    brx = xp + NBHD / 2
    bry = yp - NBHD / 2
    cx = jnp.floor((xq - tlx) / NBHD * GRID)
    cy = jnp.floor((tly - yq) / NBHD * GRID)
    cell = (cx + cy * GRID).astype(jnp.int32)
    oob_x = (xq >= brx) | (xq <= tlx)
    oob_y = (yq >= tly) | (yq <= bry)
    pp = jax.lax.broadcasted_iota(jnp.int32, (PEDS, PEDS), 0)
    qq = jax.lax.broadcasted_iota(jnp.int32, (PEDS, PEDS), 1)
    valid = jnp.logical_not(oob_x | oob_y) & (pp != qq)
    c_iota = jax.lax.broadcasted_iota(jnp.int32, (PEDS, G2, PEDS), 1)
    hit = (cell[:, None, :] == c_iota) & valid[:, None, :]
    m = jnp.where(hit, 1.0, 0.0).reshape(PEDS * G2, PEDS)
    pool = jnp.dot(m, h_ref[...], preferred_element_type=jnp.float32)
    out_ref[...] = pool.astype(jnp.bfloat16)


def _proj_body(x_ref, w_ref, y_ref, s_ref, s2_ref):
    i = pl.program_id(0)
    y = jax.lax.dot_general(
        x_ref[...], w_ref[...], (((1,), (1,)), ((), ())),
        preferred_element_type=jnp.float32)
    y_ref[...] = y
    ps = jnp.sum(y, axis=0, keepdims=True)
    ps2 = jnp.sum(y * y, axis=0, keepdims=True)

    @pl.when(i == 0)
    def _():
        s_ref[...] = ps
        s2_ref[...] = ps2

    @pl.when(i != 0)
    def _():
        s_ref[...] += ps
        s2_ref[...] += ps2


def _bn_body(y_ref, s_ref, s2_ref, g_ref, bt_ref, out_ref):
    mean = s_ref[...] * (1.0 / BATCH)
    ex2 = s2_ref[...] * (1.0 / BATCH)
    var = ex2 - mean * mean
    inv = jax.lax.rsqrt(var + 1e-5)
    yn = (y_ref[...] - mean) * inv * g_ref[...] + bt_ref[...]
    out_ref[...] = jnp.maximum(yn, 0.0)


@functools.partial(jax.jit, static_argnames=())
def kernel(h_states, seq_start_end, end_pos, rel_pos, W, b, gamma, beta):
    del seq_start_end, rel_pos
    h_flat = h_states.reshape(BATCH, H_DIM)
    xq = end_pos[:, 0].reshape(NUM_SEQS, 1, PEDS)
    yq = end_pos[:, 1].reshape(NUM_SEQS, 1, PEDS)
    xp = end_pos[:, 0].reshape(BATCH, 1)
    yp = end_pos[:, 1].reshape(BATCH, 1)

    pool = pl.pallas_call(
        _pool_body,
        grid=(NUM_SEQS,),
        in_specs=[
            pl.BlockSpec((PEDS, H_DIM), lambda i: (i, 0)),
            pl.BlockSpec((1, 1, PEDS), lambda i: (i, 0, 0)),
            pl.BlockSpec((1, 1, PEDS), lambda i: (i, 0, 0)),
            pl.BlockSpec((PEDS, 1), lambda i: (i, 0)),
            pl.BlockSpec((PEDS, 1), lambda i: (i, 0)),
        ],
        out_specs=pl.BlockSpec((PEDS * G2, H_DIM), lambda i: (i, 0)),
        out_shape=jax.ShapeDtypeStruct((BATCH * G2, H_DIM), jnp.bfloat16),
    )(h_flat, xq, yq, xp, yp)
    pool_h = pool.reshape(BATCH, G2 * H_DIM)

    y_raw, s, s2 = pl.pallas_call(
        _proj_body,
        grid=(BATCH // MBLK,),
        in_specs=[
            pl.BlockSpec((MBLK, G2 * H_DIM), lambda i: (i, 0)),
            pl.BlockSpec((BOTTLENECK, G2 * H_DIM), lambda i: (0, 0)),
        ],
        out_specs=[
            pl.BlockSpec((MBLK, BOTTLENECK), lambda i: (i, 0)),
            pl.BlockSpec((1, BOTTLENECK), lambda i: (0, 0)),
            pl.BlockSpec((1, BOTTLENECK), lambda i: (0, 0)),
        ],
        out_shape=[
            jax.ShapeDtypeStruct((BATCH, BOTTLENECK), jnp.float32),
            jax.ShapeDtypeStruct((1, BOTTLENECK), jnp.float32),
            jax.ShapeDtypeStruct((1, BOTTLENECK), jnp.float32),
        ],
    )(pool_h, W.astype(jnp.bfloat16))

    out = pl.pallas_call(
        _bn_body,
        grid=(BATCH // MBLK,),
        in_specs=[
            pl.BlockSpec((MBLK, BOTTLENECK), lambda i: (i, 0)),
            pl.BlockSpec((1, BOTTLENECK), lambda i: (0, 0)),
            pl.BlockSpec((1, BOTTLENECK), lambda i: (0, 0)),
            pl.BlockSpec((1, BOTTLENECK), lambda i: (0, 0)),
            pl.BlockSpec((1, BOTTLENECK), lambda i: (0, 0)),
        ],
        out_specs=pl.BlockSpec((MBLK, BOTTLENECK), lambda i: (i, 0)),
        out_shape=jax.ShapeDtypeStruct((BATCH, BOTTLENECK), jnp.float32),
    )(y_raw, s, s2, gamma.reshape(1, BOTTLENECK), beta.reshape(1, BOTTLENECK))
    return out


# speedup vs baseline: 13.0524x; 1.3038x over previous
import functools

import jax
import jax.numpy as jnp
from jax.experimental import pallas as pl
from jax.experimental.pallas import tpu as pltpu

H_DIM = 64
GRID = 8
G2 = GRID * GRID
NBHD = 2.0
BOTTLENECK = 1024
NUM_SEQS = 32
PEDS = 128
BATCH = NUM_SEQS * PEDS
MBLK = 512
CPACK = 2


def _pool_body(h_ref, xq_ref, yq_ref, xp_ref, yp_ref, out_ref, hd_ref):
    i = pl.program_id(0)

    @pl.when(i == 0)
    def _():
        hd_ref[...] = jnp.zeros_like(hd_ref)

    xp = xp_ref[...]
    yp = yp_ref[...]
    xq = xq_ref[0]
    yq = yq_ref[0]
    tlx = xp - NBHD / 2
    tly = yp + NBHD / 2
    brx = xp + NBHD / 2
    bry = yp - NBHD / 2
    cx = jnp.floor((xq - tlx) / NBHD * GRID)
    cy = jnp.floor((tly - yq) / NBHD * GRID)
    cell = (cx + cy * GRID).astype(jnp.int32)
    oob_x = (xq >= brx) | (xq <= tlx)
    oob_y = (yq >= tly) | (yq <= bry)
    pp = jax.lax.broadcasted_iota(jnp.int32, (PEDS, PEDS), 0)
    qq = jax.lax.broadcasted_iota(jnp.int32, (PEDS, PEDS), 1)
    valid = jnp.logical_not(oob_x | oob_y) & (pp != qq)
    ncg = G2 // CPACK
    cg = jax.lax.broadcasted_iota(jnp.int32, (PEDS, ncg, PEDS), 1)
    cell3 = cell[:, None, :]
    valid3 = valid[:, None, :]
    m_even = jnp.where((cell3 == cg * CPACK) & valid3, 1.0, 0.0).reshape(
        PEDS * ncg, PEDS)
    m_odd = jnp.where((cell3 == cg * CPACK + 1) & valid3, 1.0, 0.0).reshape(
        PEDS * ncg, PEDS)
    m = jnp.concatenate([m_even, m_odd], axis=1).astype(jnp.bfloat16)
    hb = h_ref[...].astype(jnp.bfloat16)
    hd_ref[0:PEDS, 0:H_DIM] = hb
    hd_ref[PEDS:2 * PEDS, H_DIM:2 * H_DIM] = hb
    pool = jnp.dot(m, hd_ref[...], preferred_element_type=jnp.float32)
    out_ref[...] = pool.astype(jnp.bfloat16)


def _proj_body(x_ref, w_ref, y_ref, s_ref, s2_ref, wb_ref):
    i = pl.program_id(0)

    @pl.when(i == 0)
    def _():
        wb_ref[...] = w_ref[...].astype(jnp.bfloat16)

    y = jax.lax.dot_general(
        x_ref[...], wb_ref[...], (((1,), (1,)), ((), ())),
        preferred_element_type=jnp.float32)
    y_ref[...] = y
    ps = jnp.sum(y, axis=0, keepdims=True)
    ps2 = jnp.sum(y * y, axis=0, keepdims=True)

    @pl.when(i == 0)
    def _():
        s_ref[...] = ps
        s2_ref[...] = ps2

    @pl.when(i != 0)
    def _():
        s_ref[...] += ps
        s2_ref[...] += ps2


def _bn_body(y_ref, s_ref, s2_ref, g_ref, bt_ref, out_ref):
    mean = s_ref[...] * (1.0 / BATCH)
    ex2 = s2_ref[...] * (1.0 / BATCH)
    var = ex2 - mean * mean
    inv = jax.lax.rsqrt(var + 1e-5)
    yn = (y_ref[...] - mean) * inv * g_ref[...] + bt_ref[...]
    out_ref[...] = jnp.maximum(yn, 0.0)


@functools.partial(jax.jit, static_argnames=())
def kernel(h_states, seq_start_end, end_pos, rel_pos, W, b, gamma, beta):
    del seq_start_end, rel_pos
    h_flat = h_states.reshape(BATCH, H_DIM)
    xq = end_pos[:, 0].reshape(NUM_SEQS, 1, PEDS)
    yq = end_pos[:, 1].reshape(NUM_SEQS, 1, PEDS)
    xp = end_pos[:, 0].reshape(BATCH, 1)
    yp = end_pos[:, 1].reshape(BATCH, 1)

    rows = PEDS * G2 // CPACK
    pool = pl.pallas_call(
        _pool_body,
        grid=(NUM_SEQS,),
        in_specs=[
            pl.BlockSpec((PEDS, H_DIM), lambda i: (i, 0)),
            pl.BlockSpec((1, 1, PEDS), lambda i: (i, 0, 0)),
            pl.BlockSpec((1, 1, PEDS), lambda i: (i, 0, 0)),
            pl.BlockSpec((PEDS, 1), lambda i: (i, 0)),
            pl.BlockSpec((PEDS, 1), lambda i: (i, 0)),
        ],
        out_specs=pl.BlockSpec((rows, CPACK * H_DIM), lambda i: (i, 0)),
        out_shape=jax.ShapeDtypeStruct((NUM_SEQS * rows, CPACK * H_DIM),
                                       jnp.bfloat16),
        scratch_shapes=[pltpu.VMEM((CPACK * PEDS, CPACK * H_DIM),
                                   jnp.bfloat16)],
    )(h_flat, xq, yq, xp, yp)
    pool_h = pool.reshape(BATCH, G2 * H_DIM)

    y_raw, s, s2 = pl.pallas_call(
        _proj_body,
        grid=(BATCH // MBLK,),
        in_specs=[
            pl.BlockSpec((MBLK, G2 * H_DIM), lambda i: (i, 0)),
            pl.BlockSpec((BOTTLENECK, G2 * H_DIM), lambda i: (0, 0)),
        ],
        out_specs=[
            pl.BlockSpec((MBLK, BOTTLENECK), lambda i: (i, 0)),
            pl.BlockSpec((1, BOTTLENECK), lambda i: (0, 0)),
            pl.BlockSpec((1, BOTTLENECK), lambda i: (0, 0)),
        ],
        out_shape=[
            jax.ShapeDtypeStruct((BATCH, BOTTLENECK), jnp.float32),
            jax.ShapeDtypeStruct((1, BOTTLENECK), jnp.float32),
            jax.ShapeDtypeStruct((1, BOTTLENECK), jnp.float32),
        ],
        scratch_shapes=[pltpu.VMEM((BOTTLENECK, G2 * H_DIM), jnp.bfloat16)],
    )(pool_h, W)

    out = pl.pallas_call(
        _bn_body,
        grid=(BATCH // MBLK,),
        in_specs=[
            pl.BlockSpec((MBLK, BOTTLENECK), lambda i: (i, 0)),
            pl.BlockSpec((1, BOTTLENECK), lambda i: (0, 0)),
            pl.BlockSpec((1, BOTTLENECK), lambda i: (0, 0)),
            pl.BlockSpec((1, BOTTLENECK), lambda i: (0, 0)),
            pl.BlockSpec((1, BOTTLENECK), lambda i: (0, 0)),
        ],
        out_specs=pl.BlockSpec((MBLK, BOTTLENECK), lambda i: (i, 0)),
        out_shape=jax.ShapeDtypeStruct((BATCH, BOTTLENECK), jnp.float32),
    )(y_raw, s, s2, gamma.reshape(1, BOTTLENECK), beta.reshape(1, BOTTLENECK))
    return out


# speedup vs baseline: 13.3645x; 1.0239x over previous
import functools

import jax
import jax.numpy as jnp
from jax.experimental import pallas as pl
from jax.experimental.pallas import tpu as pltpu

H_DIM = 64
GRID = 8
G2 = GRID * GRID
NBHD = 2.0
BOTTLENECK = 1024
NUM_SEQS = 32
PEDS = 128
BATCH = NUM_SEQS * PEDS
MBLK = 1024
CPACK = 2


def _pool_body(h_ref, xq_ref, yq_ref, xp_ref, yp_ref, out_ref, hd_ref):
    i = pl.program_id(0)

    @pl.when(i == 0)
    def _():
        hd_ref[...] = jnp.zeros_like(hd_ref)

    xp = xp_ref[...]
    yp = yp_ref[...]
    xq = xq_ref[0]
    yq = yq_ref[0]
    tlx = xp - NBHD / 2
    tly = yp + NBHD / 2
    brx = xp + NBHD / 2
    bry = yp - NBHD / 2
    cx = jnp.floor((xq - tlx) / NBHD * GRID)
    cy = jnp.floor((tly - yq) / NBHD * GRID)
    cell = (cx + cy * GRID).astype(jnp.int32)
    oob_x = (xq >= brx) | (xq <= tlx)
    oob_y = (yq >= tly) | (yq <= bry)
    pp = jax.lax.broadcasted_iota(jnp.int32, (PEDS, PEDS), 0)
    qq = jax.lax.broadcasted_iota(jnp.int32, (PEDS, PEDS), 1)
    valid = jnp.logical_not(oob_x | oob_y) & (pp != qq)
    ncg = G2 // CPACK
    cg = jax.lax.broadcasted_iota(jnp.int32, (PEDS, ncg, PEDS), 1)
    cell3 = cell[:, None, :]
    valid3 = valid[:, None, :]
    m_even = jnp.where((cell3 == cg * CPACK) & valid3, 1.0, 0.0).reshape(
        PEDS * ncg, PEDS)
    m_odd = jnp.where((cell3 == cg * CPACK + 1) & valid3, 1.0, 0.0).reshape(
        PEDS * ncg, PEDS)
    m = jnp.concatenate([m_even, m_odd], axis=1).astype(jnp.bfloat16)
    hb = h_ref[...].astype(jnp.bfloat16)
    hd_ref[0:PEDS, 0:H_DIM] = hb
    hd_ref[PEDS:2 * PEDS, H_DIM:2 * H_DIM] = hb
    pool = jnp.dot(m, hd_ref[...], preferred_element_type=jnp.float32)
    out_ref[...] = pool.astype(jnp.bfloat16)


def _proj_body(x_ref, w_ref, y_ref, s_ref, s2_ref, wb_ref):
    i = pl.program_id(0)

    @pl.when(i == 0)
    def _():
        wb_ref[...] = w_ref[...].astype(jnp.bfloat16)

    y = jax.lax.dot_general(
        x_ref[...], wb_ref[...], (((1,), (1,)), ((), ())),
        preferred_element_type=jnp.float32)
    y_ref[...] = y.astype(jnp.bfloat16)
    ps = jnp.sum(y, axis=0, keepdims=True)
    ps2 = jnp.sum(y * y, axis=0, keepdims=True)

    @pl.when(i == 0)
    def _():
        s_ref[...] = ps
        s2_ref[...] = ps2

    @pl.when(i != 0)
    def _():
        s_ref[...] += ps
        s2_ref[...] += ps2


def _bn_body(y_ref, s_ref, s2_ref, g_ref, bt_ref, out_ref):
    mean = s_ref[...] * (1.0 / BATCH)
    ex2 = s2_ref[...] * (1.0 / BATCH)
    var = ex2 - mean * mean
    inv = jax.lax.rsqrt(var + 1e-5)
    yn = (y_ref[...].astype(jnp.float32) - mean) * inv * g_ref[...] + bt_ref[...]
    out_ref[...] = jnp.maximum(yn, 0.0)


@functools.partial(jax.jit, static_argnames=())
def kernel(h_states, seq_start_end, end_pos, rel_pos, W, b, gamma, beta):
    del seq_start_end, rel_pos
    h_flat = h_states.reshape(BATCH, H_DIM)
    xq = end_pos[:, 0].reshape(NUM_SEQS, 1, PEDS)
    yq = end_pos[:, 1].reshape(NUM_SEQS, 1, PEDS)
    xp = end_pos[:, 0].reshape(BATCH, 1)
    yp = end_pos[:, 1].reshape(BATCH, 1)

    rows = PEDS * G2 // CPACK
    pool = pl.pallas_call(
        _pool_body,
        grid=(NUM_SEQS,),
        in_specs=[
            pl.BlockSpec((PEDS, H_DIM), lambda i: (i, 0)),
            pl.BlockSpec((1, 1, PEDS), lambda i: (i, 0, 0)),
            pl.BlockSpec((1, 1, PEDS), lambda i: (i, 0, 0)),
            pl.BlockSpec((PEDS, 1), lambda i: (i, 0)),
            pl.BlockSpec((PEDS, 1), lambda i: (i, 0)),
        ],
        out_specs=pl.BlockSpec((rows, CPACK * H_DIM), lambda i: (i, 0)),
        out_shape=jax.ShapeDtypeStruct((NUM_SEQS * rows, CPACK * H_DIM),
                                       jnp.bfloat16),
        scratch_shapes=[pltpu.VMEM((CPACK * PEDS, CPACK * H_DIM),
                                   jnp.bfloat16)],
    )(h_flat, xq, yq, xp, yp)
    pool_h = pool.reshape(BATCH, G2 * H_DIM)

    y_raw, s, s2 = pl.pallas_call(
        _proj_body,
        grid=(BATCH // MBLK,),
        in_specs=[
            pl.BlockSpec((MBLK, G2 * H_DIM), lambda i: (i, 0)),
            pl.BlockSpec((BOTTLENECK, G2 * H_DIM), lambda i: (0, 0)),
        ],
        out_specs=[
            pl.BlockSpec((MBLK, BOTTLENECK), lambda i: (i, 0)),
            pl.BlockSpec((1, BOTTLENECK), lambda i: (0, 0)),
            pl.BlockSpec((1, BOTTLENECK), lambda i: (0, 0)),
        ],
        out_shape=[
            jax.ShapeDtypeStruct((BATCH, BOTTLENECK), jnp.bfloat16),
            jax.ShapeDtypeStruct((1, BOTTLENECK), jnp.float32),
            jax.ShapeDtypeStruct((1, BOTTLENECK), jnp.float32),
        ],
        scratch_shapes=[pltpu.VMEM((BOTTLENECK, G2 * H_DIM), jnp.bfloat16)],
    )(pool_h, W)

    out = pl.pallas_call(
        _bn_body,
        grid=(BATCH // MBLK,),
        in_specs=[
            pl.BlockSpec((MBLK, BOTTLENECK), lambda i: (i, 0)),
            pl.BlockSpec((1, BOTTLENECK), lambda i: (0, 0)),
            pl.BlockSpec((1, BOTTLENECK), lambda i: (0, 0)),
            pl.BlockSpec((1, BOTTLENECK), lambda i: (0, 0)),
            pl.BlockSpec((1, BOTTLENECK), lambda i: (0, 0)),
        ],
        out_specs=pl.BlockSpec((MBLK, BOTTLENECK), lambda i: (i, 0)),
        out_shape=jax.ShapeDtypeStruct((BATCH, BOTTLENECK), jnp.float32),
    )(y_raw, s, s2, gamma.reshape(1, BOTTLENECK), beta.reshape(1, BOTTLENECK))
    return out
